# baseline (speedup 1.0000x reference)
import functools

import jax
import jax.numpy as jnp
from jax import lax
from jax.experimental import pallas as pl
from jax.experimental.pallas import tpu as pltpu
from jax.experimental.pallas import tpu_sc as plsc

B = 16384
GMF_DIM = 16
MLP_DIM = 32
NC = 2
NS = 16
NW = NC * NS
BPW = B // NW
CHUNK = 128
NCHUNK = BPW // CHUNK
IDX_ROWS = B // CHUNK


def _sc_gather_body(u2, i2, gue, gie, gub, gib, mue, mie,
                    gu_o, gi_o, bu_o, bi_o, mu_o, mi_o,
                    idx_u, idx_i, gu_v, gi_v, bu_v, bi_v, mu_v, mi_v, sem):
    c = lax.axis_index("c")
    s = lax.axis_index("s")
    wid = s * NC + c
    r0 = wid * NCHUNK
    b0 = wid * BPW

    pltpu.sync_copy(u2.at[pl.ds(r0, NCHUNK)], idx_u)
    pltpu.sync_copy(i2.at[pl.ds(r0, NCHUNK)], idx_i)

    copies = []
    for j in range(NCHUNK):
        sl = pl.ds(j * CHUNK, CHUNK)
        copies.append(pltpu.async_copy(gue.at[idx_u.at[j]], gu_v.at[sl], sem))
        copies.append(pltpu.async_copy(gie.at[idx_i.at[j]], gi_v.at[sl], sem))
        copies.append(pltpu.async_copy(mue.at[idx_u.at[j]], mu_v.at[sl], sem))
        copies.append(pltpu.async_copy(mie.at[idx_i.at[j]], mi_v.at[sl], sem))
        copies.append(pltpu.async_copy(gub.at[idx_u.at[j]], bu_v.at[j], sem))
        copies.append(pltpu.async_copy(gib.at[idx_i.at[j]], bi_v.at[j], sem))
    for cp in copies:
        cp.wait()

    pltpu.sync_copy(gu_v, gu_o.at[pl.ds(b0, BPW)])
    pltpu.sync_copy(gi_v, gi_o.at[pl.ds(b0, BPW)])
    pltpu.sync_copy(mu_v, mu_o.at[pl.ds(b0, BPW)])
    pltpu.sync_copy(mi_v, mi_o.at[pl.ds(b0, BPW)])
    pltpu.sync_copy(bu_v, bu_o.at[pl.ds(r0, NCHUNK)])
    pltpu.sync_copy(bi_v, bi_o.at[pl.ds(r0, NCHUNK)])


@jax.jit
def _sc_gather(u2, i2, gue, gie, gub, gib, mue, mie):
    mesh = plsc.VectorSubcoreMesh(core_axis_name="c", subcore_axis_name="s")
    f = pl.kernel(
        _sc_gather_body,
        out_type=[
            jax.ShapeDtypeStruct((B, GMF_DIM), jnp.float32),
            jax.ShapeDtypeStruct((B, GMF_DIM), jnp.float32),
            jax.ShapeDtypeStruct((IDX_ROWS, CHUNK), jnp.float32),
            jax.ShapeDtypeStruct((IDX_ROWS, CHUNK), jnp.float32),
            jax.ShapeDtypeStruct((B, MLP_DIM), jnp.float32),
            jax.ShapeDtypeStruct((B, MLP_DIM), jnp.float32),
        ],
        mesh=mesh,
        scratch_types=[
            pltpu.VMEM((NCHUNK, CHUNK), jnp.int32),
            pltpu.VMEM((NCHUNK, CHUNK), jnp.int32),
            pltpu.VMEM((BPW, GMF_DIM), jnp.float32),
            pltpu.VMEM((BPW, GMF_DIM), jnp.float32),
            pltpu.VMEM((NCHUNK, CHUNK), jnp.float32),
            pltpu.VMEM((NCHUNK, CHUNK), jnp.float32),
            pltpu.VMEM((BPW, MLP_DIM), jnp.float32),
            pltpu.VMEM((BPW, MLP_DIM), jnp.float32),
            pltpu.SemaphoreType.DMA,
        ],
        compiler_params=pltpu.CompilerParams(use_tc_tiling_on_sc=False),
    )
    return f(u2, i2, gue, gie, gub, gib, mue, mie)


BLK = 2048


def _tc_body(gu, gi, bu, bi, mu, mi, w1a, w1b, b1, w2, b2, wf, bf, out):
    h = jnp.dot(mu[...], w1a[...], preferred_element_type=jnp.float32)
    h += jnp.dot(mi[...], w1b[...], preferred_element_type=jnp.float32)
    h = jnp.maximum(h + b1[...], 0.0)
    h = jnp.maximum(jnp.dot(h, w2[...], preferred_element_type=jnp.float32)
                    + b2[...], 0.0)
    gmf = (jnp.sum(gu[...] * gi[...], axis=1, keepdims=True)
           + bu[...] + bi[...])
    wfv = wf[...]
    pred = (gmf * wfv[:, 0:1]
            + jnp.sum(h * wfv[:, 1:], axis=1, keepdims=True)
            + bf[...])
    out[...] = pred


@jax.jit
def _tc_mlp(gu, gi, bu, bi, mu, mi, w1a, w1b, b1, w2, b2, wf, bf):
    grid = B // BLK
    full = lambda i: (0, 0)
    blk_row = lambda i: (i, 0)
    return pl.pallas_call(
        _tc_body,
        grid=(grid,),
        in_specs=[
            pl.BlockSpec((BLK, GMF_DIM), blk_row),
            pl.BlockSpec((BLK, GMF_DIM), blk_row),
            pl.BlockSpec((BLK, 1), blk_row),
            pl.BlockSpec((BLK, 1), blk_row),
            pl.BlockSpec((BLK, MLP_DIM), blk_row),
            pl.BlockSpec((BLK, MLP_DIM), blk_row),
            pl.BlockSpec((MLP_DIM, 32), full),
            pl.BlockSpec((MLP_DIM, 32), full),
            pl.BlockSpec((1, 32), full),
            pl.BlockSpec((32, 16), full),
            pl.BlockSpec((1, 16), full),
            pl.BlockSpec((1, 17), full),
            pl.BlockSpec((1, 1), full),
        ],
        out_specs=pl.BlockSpec((BLK, 1), blk_row),
        out_shape=jax.ShapeDtypeStruct((B, 1), jnp.float32),
    )(gu, gi, bu, bi, mu, mi, w1a, w1b, b1, w2, b2, wf, bf)


def kernel(user_ids, item_ids, gmf_user_emb, gmf_item_emb, gmf_user_bias,
           gmf_item_bias, mlp_user_emb, mlp_item_emb, W1, b1, W2, b2, Wf, bf):
    u2 = user_ids.astype(jnp.int32).reshape(IDX_ROWS, CHUNK)
    i2 = item_ids.astype(jnp.int32).reshape(IDX_ROWS, CHUNK)
    gu, gi, bu, bi, mu, mi = _sc_gather(
        u2, i2, gmf_user_emb, gmf_item_emb, gmf_user_bias, gmf_item_bias,
        mlp_user_emb, mlp_item_emb)
    pred = _tc_mlp(
        gu, gi, bu.reshape(B, 1), bi.reshape(B, 1), mu, mi,
        W1[:MLP_DIM], W1[MLP_DIM:], b1.reshape(1, -1), W2, b2.reshape(1, -1),
        Wf.reshape(1, -1), bf.reshape(1, 1))
    return pred.reshape(-1)
